# baseline (speedup 1.0000x reference)
import functools

import jax
import jax.numpy as jnp
from jax import lax
from jax.experimental import pallas as pl
from jax.experimental.pallas import tpu as pltpu
from jax.experimental.pallas import tpu_sc as plsc

F32 = jnp.float32
I32 = jnp.int32

H = 256
NW = 32
CHUNK = 128
NEG = -1e30


def _erf(x):
    s = jnp.sign(x)
    a = jnp.abs(x)
    t = 1.0 / (1.0 + 0.3275911 * a)
    poly = t * (0.254829592 + t * (-0.284496736 + t * (1.421413741
           + t * (-1.453152027 + t * 1.061405429))))
    return s * (1.0 - poly * jnp.exp(-a * a))


def _gelu(x):
    return 0.5 * x * (1.0 + _erf(x * 0.7071067811865476))


def _enc1_body(embT_ref, op_ref, nfT_ref, w1_ref, e1T_ref):
    embT = embT_ref[...]
    nrm = jnp.sqrt(jnp.sum(embT * embT, axis=0, keepdims=True))
    scale = jnp.minimum(1.0, 1.0 / jnp.maximum(nrm, 1e-07))
    embT = embT * scale
    op_row = op_ref[0:1, :]
    oh = (lax.broadcasted_iota(I32, (embT.shape[1], op_row.shape[1]), 0)
          == op_row).astype(F32)
    x0_emb = lax.dot_general(embT, oh, (((1,), (0,)), ((), ())),
                             preferred_element_type=F32)
    x0 = jnp.concatenate([x0_emb, nfT_ref[...]], axis=0)
    e1T_ref[...] = lax.dot_general(w1_ref[...], x0, (((0,), (0,)), ((), ())),
                                   preferred_element_type=F32)


def _enc1(embT, op2d, nfTn, w1, npad, bn):
    grid = npad // bn
    return pl.pallas_call(
        _enc1_body,
        grid=(grid,),
        in_specs=[
            pl.BlockSpec(embT.shape, lambda i: (0, 0)),
            pl.BlockSpec((8, bn), lambda i: (0, i)),
            pl.BlockSpec((140, bn), lambda i: (0, i)),
            pl.BlockSpec(w1.shape, lambda i: (0, 0)),
        ],
        out_specs=pl.BlockSpec((H, bn), lambda i: (0, i)),
        out_shape=jax.ShapeDtypeStruct((H, npad), F32),
    )(embT, op2d, nfTn, w1)


def _mmT_body(w_ref, xT_ref, oT_ref):
    oT_ref[...] = lax.dot_general(w_ref[...], xT_ref[...],
                                  (((0,), (0,)), ((), ())),
                                  preferred_element_type=F32)


def _mmT(w, xT, npad, bn):
    grid = npad // bn
    return pl.pallas_call(
        _mmT_body,
        grid=(grid,),
        in_specs=[
            pl.BlockSpec(w.shape, lambda i: (0, 0)),
            pl.BlockSpec((w.shape[0], bn), lambda i: (0, i)),
        ],
        out_specs=pl.BlockSpec((w.shape[1], bn), lambda i: (0, i)),
        out_shape=jax.ShapeDtypeStruct((w.shape[1], npad), F32),
    )(w, xT)


def _gat_mm_body(xT_ref, wl_ref, wr_ref, bl_ref, br_ref, blc_ref,
                 xl_ref, xr_ref, xlT_ref):
    xTb = xT_ref[...]
    wl = wl_ref[...]
    wr = wr_ref[...]
    xl_ref[...] = lax.dot_general(xTb, wl, (((0,), (0,)), ((), ())),
                                  preferred_element_type=F32) + bl_ref[...]
    xr_ref[...] = lax.dot_general(xTb, wr, (((0,), (0,)), ((), ())),
                                  preferred_element_type=F32) + br_ref[...]
    xlT_ref[...] = lax.dot_general(wl, xTb, (((0,), (0,)), ((), ())),
                                   preferred_element_type=F32) + blc_ref[...]


def _gat_mm(xT, wl, wr, bl, br, npad, bn):
    grid = npad // bn
    return pl.pallas_call(
        _gat_mm_body,
        grid=(grid,),
        in_specs=[
            pl.BlockSpec((H, bn), lambda i: (0, i)),
            pl.BlockSpec((H, H), lambda i: (0, 0)),
            pl.BlockSpec((H, H), lambda i: (0, 0)),
            pl.BlockSpec((1, H), lambda i: (0, 0)),
            pl.BlockSpec((1, H), lambda i: (0, 0)),
            pl.BlockSpec((H, 1), lambda i: (0, 0)),
        ],
        out_specs=[
            pl.BlockSpec((bn, H), lambda i: (i, 0)),
            pl.BlockSpec((bn, H), lambda i: (i, 0)),
            pl.BlockSpec((H, bn), lambda i: (0, i)),
        ],
        out_shape=[
            jax.ShapeDtypeStruct((npad, H), F32),
            jax.ShapeDtypeStruct((npad, H), F32),
            jax.ShapeDtypeStruct((H, npad), F32),
        ],
    )(xT, wl, wr, bl.reshape(1, H), br.reshape(1, H), bl.reshape(H, 1))


def _ng_body(n_real, yT_ref, oT_ref):
    y = yT_ref[...]
    npad = y.shape[1]
    mask = (lax.broadcasted_iota(I32, (1, npad), 1) < n_real).astype(F32)
    s = jnp.sum(y * mask, axis=1, keepdims=True)
    m = s * (1.0 / n_real)
    d = (y - m) * mask
    v = jnp.sum(d * d, axis=1, keepdims=True) * (1.0 / n_real)
    oT_ref[...] = _gelu((y - m) / jnp.sqrt(v + 1e-05))


def _ng(yT, n_real, npad, br=64):
    grid = H // br
    return pl.pallas_call(
        functools.partial(_ng_body, n_real),
        grid=(grid,),
        in_specs=[pl.BlockSpec((br, npad), lambda i: (i, 0))],
        out_specs=pl.BlockSpec((br, npad), lambda i: (i, 0)),
        out_shape=jax.ShapeDtypeStruct((H, npad), F32),
    )(yT)


def _ng_pool_body(n_real, yT_ref, oT_ref, pool_ref):
    y = yT_ref[...]
    npad = y.shape[1]
    maskb = lax.broadcasted_iota(I32, (1, npad), 1) < n_real
    mask = maskb.astype(F32)
    s = jnp.sum(y * mask, axis=1, keepdims=True)
    m = s * (1.0 / n_real)
    d = (y - m) * mask
    v = jnp.sum(d * d, axis=1, keepdims=True) * (1.0 / n_real)
    out = _gelu((y - m) / jnp.sqrt(v + 1e-05))
    oT_ref[...] = out
    pmean = jnp.sum(out * mask, axis=1, keepdims=True) * (1.0 / n_real)
    pmax = jnp.max(jnp.where(maskb, out, NEG), axis=1, keepdims=True)
    pool_ref[...] = jnp.broadcast_to(pmean + pmax, (out.shape[0], 128))


def _ng_pool(yT, n_real, npad, br=64):
    grid = H // br
    return pl.pallas_call(
        functools.partial(_ng_pool_body, n_real),
        grid=(grid,),
        in_specs=[pl.BlockSpec((br, npad), lambda i: (i, 0))],
        out_specs=[
            pl.BlockSpec((br, npad), lambda i: (i, 0)),
            pl.BlockSpec((br, 128), lambda i: (i, 0)),
        ],
        out_shape=[
            jax.ShapeDtypeStruct((H, npad), F32),
            jax.ShapeDtypeStruct((H, 128), F32),
        ],
    )(yT)


def _head_body(xl_ref, w1_ref, w2_ref, pw_ref, pb_ref, out_ref):
    xl = xl_ref[...]
    h = xl @ w1_ref[...]
    m = jnp.mean(h, axis=0, keepdims=True)
    v = jnp.mean((h - m) ** 2, axis=0, keepdims=True)
    h = _gelu((h - m) / jnp.sqrt(v + 1e-05))
    h2 = h @ w2_ref[...]
    m2 = jnp.mean(h2, axis=0, keepdims=True)
    v2 = jnp.mean((h2 - m2) ** 2, axis=0, keepdims=True)
    h2 = _gelu((h2 - m2) / jnp.sqrt(v2 + 1e-05))
    out_ref[...] = h2 @ pw_ref[...] + pb_ref[0, 0]


def _head(xl, w1, w2, pw, pb):
    nc = xl.shape[0]
    return pl.pallas_call(
        _head_body,
        out_shape=jax.ShapeDtypeStruct((nc, 1), F32),
    )(xl, w1, w2, pw.reshape(-1, 1), pb.reshape(1, 1))


def _sc_mesh():
    return plsc.VectorSubcoreMesh(core_axis_name="c", subcore_axis_name="s")


def _vrot(scratch16, x, sh):
    scratch16[...] = x
    idx = (jnp.arange(16, dtype=I32) + sh) & 15
    return plsc.load_gather(scratch16, [idx])


def _vallsum(scratch16, x):
    for sh in (8, 4, 2, 1):
        x = x + _vrot(scratch16, x, sh)
    return x


def _vallmax(scratch16, x):
    for sh in (8, 4, 2, 1):
        x = jnp.maximum(x, _vrot(scratch16, x, sh))
    return x


def _make_k1(npad, epad, e_real):
    epw = epad // NW
    cpw = epw // CHUNK

    @functools.partial(
        pl.kernel,
        mesh=_sc_mesh(),
        compiler_params=pltpu.CompilerParams(needs_layout_passes=False),
        out_type=(
            jax.ShapeDtypeStruct((epad,), F32),
            jax.ShapeDtypeStruct((NW, 16), F32),
        ),
        scratch_types=[
            pltpu.VMEM((CHUNK,), I32),
            pltpu.VMEM((CHUNK,), I32),
            pltpu.VMEM((CHUNK, H), F32),
            pltpu.VMEM((CHUNK, H), F32),
            pltpu.VMEM((CHUNK,), F32),
            pltpu.VMEM((H,), F32),
            pltpu.VMEM((16,), F32),
            pltpu.VMEM((16,), F32),
            pltpu.SemaphoreType.DMA,
            pltpu.SemaphoreType.DMA,
        ],
    )
    def k1(xl_hbm, xr_hbm, srcs_hbm, dsts_hbm, att_hbm, lg_hbm, wmax_hbm,
           idx_s, idx_d, rows_l, rows_r, lg_v, att_v, wv, bounce,
           sem1, sem2):
        wid = lax.axis_index("s") * 2 + lax.axis_index("c")
        base_w = wid * epw
        pltpu.sync_copy(att_hbm, att_v)
        att_regs = [att_v[pl.ds(16 * j, 16)] for j in range(16)]
        lane0 = jnp.arange(16, dtype=I32) == 0

        def chunk_body(c, mx):
            cb = base_w + c * CHUNK
            pltpu.sync_copy(srcs_hbm.at[pl.ds(cb, CHUNK)], idx_s)
            pltpu.sync_copy(dsts_hbm.at[pl.ds(cb, CHUNK)], idx_d)
            cp1 = pltpu.async_copy(xl_hbm.at[idx_s], rows_l, sem1)
            cp2 = pltpu.async_copy(xr_hbm.at[idx_d], rows_r, sem2)
            cp1.wait()
            cp2.wait()

            def edge_body(i, mx_in):
                acc = jnp.zeros((16,), F32)
                for j in range(16):
                    t = (rows_l[i, pl.ds(16 * j, 16)]
                         + rows_r[i, pl.ds(16 * j, 16)])
                    t = jnp.where(t > 0, t, 0.2 * t)
                    acc = acc + att_regs[j] * t
                lgs = _vallsum(bounce, acc)
                lgs = jnp.where(jnp.full((16,), cb + i, I32)
                                < jnp.full((16,), e_real, I32),
                                lgs, jnp.full((16,), NEG, F32))
                plsc.store_scatter(lg_v, [jnp.full((16,), i, I32)],
                                   lgs, mask=lane0)
                return jnp.maximum(mx_in, lgs)

            mx = lax.fori_loop(0, CHUNK, edge_body, mx)
            pltpu.sync_copy(lg_v, lg_hbm.at[pl.ds(cb, CHUNK)])
            return mx

        mx = lax.fori_loop(0, cpw, chunk_body, jnp.full((16,), NEG, F32))
        wv[...] = mx
        pltpu.sync_copy(wv, wmax_hbm.at[wid])

    return k1


def _make_k2(npad, epad):
    epw = epad // NW
    cpw = epw // CHUNK
    nseg = npad // 16

    @functools.partial(
        pl.kernel,
        mesh=_sc_mesh(),
        compiler_params=pltpu.CompilerParams(needs_layout_passes=False),
        out_type=jax.ShapeDtypeStruct((2, npad), F32),
        scratch_types=[
            pltpu.VMEM((npad,), F32),
            pltpu.VMEM((CHUNK,), I32),
            pltpu.VMEM((CHUNK,), F32),
            pltpu.VMEM((NW, 16), F32),
            pltpu.VMEM((nseg,), F32),
            pltpu.VMEM((nseg,), F32),
            pltpu.VMEM((16,), F32),
            pltpu.VMEM_SHARED((16, npad), F32),
        ],
    )
    def k2(lg_hbm, dsts_hbm, wmax_hbm, den_hbm,
           den_v, idx_d, lg_v, wmax_v, tmp_v, acc_v, bounce, den_sh):
        cid = lax.axis_index("c")
        sid = lax.axis_index("s")
        wid = sid * 2 + cid
        pltpu.sync_copy(wmax_hbm, wmax_v)
        g = jnp.full((16,), NEG, F32)
        for w in range(NW):
            g = jnp.maximum(g, wmax_v[w, :])
        gmax = _vallmax(bounce, g)

        def zero_body(i, _):
            den_v[pl.ds(i * 16, 16)] = jnp.zeros((16,), F32)
            return 0

        lax.fori_loop(0, npad // 16, zero_body, 0)

        def chunk_body(c, _):
            cb = wid * epw + c * CHUNK
            pltpu.sync_copy(dsts_hbm.at[pl.ds(cb, CHUNK)], idx_d)
            pltpu.sync_copy(lg_hbm.at[pl.ds(cb, CHUNK)], lg_v)
            for k in range(CHUNK // 16):
                d16 = idx_d[pl.ds(16 * k, 16)]
                ex = jnp.exp(lg_v[pl.ds(16 * k, 16)] - gmax)
                plsc.addupdate_scatter(den_v, [d16], ex)
            return 0

        lax.fori_loop(0, cpw, chunk_body, 0)

        pltpu.sync_copy(den_v, den_sh.at[sid])
        plsc.subcore_barrier()
        nb = sid * nseg

        def acc_zero(i, _):
            acc_v[pl.ds(i * 16, 16)] = jnp.zeros((16,), F32)
            return 0

        lax.fori_loop(0, nseg // 16, acc_zero, 0)
        for w in range(16):
            pltpu.sync_copy(den_sh.at[w, pl.ds(nb, nseg)], tmp_v)

            def add_body(i, _):
                acc_v[pl.ds(i * 16, 16)] = (acc_v[pl.ds(i * 16, 16)]
                                            + tmp_v[pl.ds(i * 16, 16)])
                return 0

            lax.fori_loop(0, nseg // 16, add_body, 0)
        pltpu.sync_copy(acc_v, den_hbm.at[cid, pl.ds(nb, nseg)])

    return k2


def _make_k3(npad, epad):
    nchunks = epad // CHUNK
    CPT = 4

    @functools.partial(
        pl.kernel,
        mesh=_sc_mesh(),
        compiler_params=pltpu.CompilerParams(needs_layout_passes=False),
        out_type=jax.ShapeDtypeStruct((H, npad), F32),
        scratch_types=[
            pltpu.VMEM((CPT, npad), F32),
            pltpu.VMEM((CPT, npad), F32),
            pltpu.VMEM((npad,), F32),
            pltpu.VMEM((npad,), F32),
            pltpu.VMEM((CHUNK,), I32),
            pltpu.VMEM((CHUNK,), I32),
            pltpu.VMEM((CHUNK,), F32),
            pltpu.VMEM((NW, 16), F32),
            pltpu.VMEM((16,), F32),
        ],
    )
    def k3(xlT_hbm, srcs_hbm, dsts_hbm, lg_hbm, wmax_hbm, den_hbm, outT_hbm,
           xl4, acc4, den_v, den_tmp, idx_s, idx_d, lg_v, wmax_v, bounce):
        wid = lax.axis_index("s") * 2 + lax.axis_index("c")
        pltpu.sync_copy(wmax_hbm, wmax_v)
        g = jnp.full((16,), NEG, F32)
        for w in range(NW):
            g = jnp.maximum(g, wmax_v[w, :])
        gmax = _vallmax(bounce, g)

        pltpu.sync_copy(den_hbm.at[0], den_v)
        pltpu.sync_copy(den_hbm.at[1], den_tmp)

        def den_add(i, _):
            den_v[pl.ds(i * 16, 16)] = (den_v[pl.ds(i * 16, 16)]
                                        + den_tmp[pl.ds(i * 16, 16)])
            return 0

        lax.fori_loop(0, npad // 16, den_add, 0)

        for rnd in range(H // (NW * CPT)):
            cols = (rnd * NW + wid) * CPT

            for j in range(CPT):
                pltpu.sync_copy(xlT_hbm.at[cols + j], xl4.at[j])

            def acc_zero(i, _):
                for j in range(CPT):
                    acc4[j, pl.ds(i * 16, 16)] = jnp.zeros((16,), F32)
                return 0

            lax.fori_loop(0, npad // 16, acc_zero, 0)

            def chunk_body(c, _):
                cb = c * CHUNK
                pltpu.sync_copy(srcs_hbm.at[pl.ds(cb, CHUNK)], idx_s)
                pltpu.sync_copy(dsts_hbm.at[pl.ds(cb, CHUNK)], idx_d)
                pltpu.sync_copy(lg_hbm.at[pl.ds(cb, CHUNK)], lg_v)
                for k in range(CHUNK // 16):
                    s16 = idx_s[pl.ds(16 * k, 16)]
                    d16 = idx_d[pl.ds(16 * k, 16)]
                    ex = jnp.exp(lg_v[pl.ds(16 * k, 16)] - gmax)
                    dd = plsc.load_gather(den_v, [d16])
                    a16 = ex / (dd + 1e-16)
                    for j in range(CPT):
                        j16 = jnp.full((16,), j, I32)
                        v = plsc.load_gather(xl4, [j16, s16])
                        plsc.addupdate_scatter(acc4, [j16, d16], a16 * v)
                return 0

            lax.fori_loop(0, nchunks, chunk_body, 0)

            for j in range(CPT):
                pltpu.sync_copy(acc4.at[j], outT_hbm.at[cols + j])

    return k3


def kernel(node_feat, node_opcode, edge_index, config_feat, params):
    n = node_feat.shape[0]
    e = edge_index.shape[1]
    npad = ((n + 1023) // 1024) * 1024
    bn = 1024
    e_real = e + n
    epad = ((e_real + NW * CHUNK - 1) // (NW * CHUNK)) * (NW * CHUNK)

    loops = jnp.arange(n, dtype=edge_index.dtype)
    zpad = jnp.zeros((epad - e_real,), I32)
    srcs = jnp.concatenate([edge_index[0], loops, zpad])
    dsts = jnp.concatenate([edge_index[1], loops, zpad])

    nfTn = ((node_feat - params['nf_mean']) / (params['nf_std'] + 0.0001)).T
    nfTn = jnp.pad(nfTn, ((0, 0), (0, npad - n)))
    op2d = jnp.broadcast_to(node_opcode[None, :], (8, n)).astype(I32)
    op2d = jnp.pad(op2d, ((0, 0), (0, npad - n)), constant_values=-1)
    embT = params['embed'].T

    e1T = _enc1(embT, op2d, nfTn, params['eW1'], npad, bn)
    x1T = _ng(e1T, n, npad)
    e2T = _mmT(params['eW2'], x1T, npad, bn)
    xT = _ng(e2T, n, npad)

    k1 = _make_k1(npad, epad, e_real)
    k2 = _make_k2(npad, epad)
    k3 = _make_k3(npad, epad)
    for li, gp in enumerate(params['gat']):
        xl, xr, xlT = _gat_mm(xT, gp['Wl'], gp['Wr'], gp['bl'], gp['br'],
                              npad, bn)
        lg, wmax = k1(xl, xr, srcs, dsts, gp['att'])
        den = k2(lg, dsts, wmax)
        outT = k3(xlT, srcs, dsts, lg, wmax, den)
        if li < 3:
            xT = _ng(outT, n, npad)
        else:
            xT, pool2d = _ng_pool(outT, n, npad)

    pool = pool2d[:, 0]

    cf = (config_feat - params['cf_mean']) / (params['cf_std'] + 0.0001)
    xl_head = jnp.concatenate(
        [cf, jnp.broadcast_to(pool[None, :], (cf.shape[0], H))], axis=1)
    runtime = _head(xl_head, params['lW1'], params['lW2'],
                    params['pW'], params['pb'])
    return runtime[:, 0]


# speedup vs baseline: 3.2543x; 3.2543x over previous
import functools

import jax
import jax.numpy as jnp
from jax import lax
from jax.experimental import pallas as pl
from jax.experimental.pallas import tpu as pltpu
from jax.experimental.pallas import tpu_sc as plsc

F32 = jnp.float32
I32 = jnp.int32

H = 256
NW = 32
CHUNK = 128
NEG = -1e30


def _erf(x):
    s = jnp.sign(x)
    a = jnp.abs(x)
    t = 1.0 / (1.0 + 0.3275911 * a)
    poly = t * (0.254829592 + t * (-0.284496736 + t * (1.421413741
           + t * (-1.453152027 + t * 1.061405429))))
    return s * (1.0 - poly * jnp.exp(-a * a))


def _gelu(x):
    return 0.5 * x * (1.0 + _erf(x * 0.7071067811865476))


def _enc1_body(embT_ref, op_ref, nfT_ref, w1_ref, e1T_ref):
    embT = embT_ref[...]
    nrm = jnp.sqrt(jnp.sum(embT * embT, axis=0, keepdims=True))
    scale = jnp.minimum(1.0, 1.0 / jnp.maximum(nrm, 1e-07))
    embT = embT * scale
    op_row = op_ref[0:1, :]
    oh = (lax.broadcasted_iota(I32, (embT.shape[1], op_row.shape[1]), 0)
          == op_row).astype(F32)
    x0_emb = lax.dot_general(embT, oh, (((1,), (0,)), ((), ())),
                             preferred_element_type=F32)
    x0 = jnp.concatenate([x0_emb, nfT_ref[...]], axis=0)
    e1T_ref[...] = lax.dot_general(w1_ref[...], x0, (((0,), (0,)), ((), ())),
                                   preferred_element_type=F32)


def _enc1(embT, op2d, nfTn, w1, npad, bn):
    grid = npad // bn
    return pl.pallas_call(
        _enc1_body,
        grid=(grid,),
        in_specs=[
            pl.BlockSpec(embT.shape, lambda i: (0, 0)),
            pl.BlockSpec((8, bn), lambda i: (0, i)),
            pl.BlockSpec((140, bn), lambda i: (0, i)),
            pl.BlockSpec(w1.shape, lambda i: (0, 0)),
        ],
        out_specs=pl.BlockSpec((H, bn), lambda i: (0, i)),
        out_shape=jax.ShapeDtypeStruct((H, npad), F32),
    )(embT, op2d, nfTn, w1)


def _mmT_body(w_ref, xT_ref, oT_ref):
    oT_ref[...] = lax.dot_general(w_ref[...], xT_ref[...],
                                  (((0,), (0,)), ((), ())),
                                  preferred_element_type=F32)


def _mmT(w, xT, npad, bn):
    grid = npad // bn
    return pl.pallas_call(
        _mmT_body,
        grid=(grid,),
        in_specs=[
            pl.BlockSpec(w.shape, lambda i: (0, 0)),
            pl.BlockSpec((w.shape[0], bn), lambda i: (0, i)),
        ],
        out_specs=pl.BlockSpec((w.shape[1], bn), lambda i: (0, i)),
        out_shape=jax.ShapeDtypeStruct((w.shape[1], npad), F32),
    )(w, xT)


def _gat_mm_body(xT_ref, wl_ref, wr_ref, bl_ref, br_ref, blc_ref,
                 xl_ref, xr_ref, xlT_ref):
    xTb = xT_ref[...]
    wl = wl_ref[...]
    wr = wr_ref[...]
    xl_ref[...] = lax.dot_general(xTb, wl, (((0,), (0,)), ((), ())),
                                  preferred_element_type=F32) + bl_ref[...]
    xr_ref[...] = lax.dot_general(xTb, wr, (((0,), (0,)), ((), ())),
                                  preferred_element_type=F32) + br_ref[...]
    xlT_ref[...] = lax.dot_general(wl, xTb, (((0,), (0,)), ((), ())),
                                   preferred_element_type=F32) + blc_ref[...]


def _gat_mm(xT, wl, wr, bl, br, npad, bn):
    grid = npad // bn
    return pl.pallas_call(
        _gat_mm_body,
        grid=(grid,),
        in_specs=[
            pl.BlockSpec((H, bn), lambda i: (0, i)),
            pl.BlockSpec((H, H), lambda i: (0, 0)),
            pl.BlockSpec((H, H), lambda i: (0, 0)),
            pl.BlockSpec((1, H), lambda i: (0, 0)),
            pl.BlockSpec((1, H), lambda i: (0, 0)),
            pl.BlockSpec((H, 1), lambda i: (0, 0)),
        ],
        out_specs=[
            pl.BlockSpec((bn, H), lambda i: (i, 0)),
            pl.BlockSpec((bn, H), lambda i: (i, 0)),
            pl.BlockSpec((H, bn), lambda i: (0, i)),
        ],
        out_shape=[
            jax.ShapeDtypeStruct((npad, H), F32),
            jax.ShapeDtypeStruct((npad, H), F32),
            jax.ShapeDtypeStruct((H, npad), F32),
        ],
    )(xT, wl, wr, bl.reshape(1, H), br.reshape(1, H), bl.reshape(H, 1))


def _ng_body(n_real, yT_ref, oT_ref):
    y = yT_ref[...]
    npad = y.shape[1]
    mask = (lax.broadcasted_iota(I32, (1, npad), 1) < n_real).astype(F32)
    s = jnp.sum(y * mask, axis=1, keepdims=True)
    m = s * (1.0 / n_real)
    d = (y - m) * mask
    v = jnp.sum(d * d, axis=1, keepdims=True) * (1.0 / n_real)
    oT_ref[...] = _gelu((y - m) / jnp.sqrt(v + 1e-05))


def _ng(yT, n_real, npad, br=64):
    grid = H // br
    return pl.pallas_call(
        functools.partial(_ng_body, n_real),
        grid=(grid,),
        in_specs=[pl.BlockSpec((br, npad), lambda i: (i, 0))],
        out_specs=pl.BlockSpec((br, npad), lambda i: (i, 0)),
        out_shape=jax.ShapeDtypeStruct((H, npad), F32),
    )(yT)


def _ng_pool_body(n_real, yT_ref, oT_ref, pool_ref):
    y = yT_ref[...]
    npad = y.shape[1]
    maskb = lax.broadcasted_iota(I32, (1, npad), 1) < n_real
    mask = maskb.astype(F32)
    s = jnp.sum(y * mask, axis=1, keepdims=True)
    m = s * (1.0 / n_real)
    d = (y - m) * mask
    v = jnp.sum(d * d, axis=1, keepdims=True) * (1.0 / n_real)
    out = _gelu((y - m) / jnp.sqrt(v + 1e-05))
    oT_ref[...] = out
    pmean = jnp.sum(out * mask, axis=1, keepdims=True) * (1.0 / n_real)
    pmax = jnp.max(jnp.where(maskb, out, NEG), axis=1, keepdims=True)
    pool_ref[...] = jnp.broadcast_to(pmean + pmax, (out.shape[0], 128))


def _ng_pool(yT, n_real, npad, br=64):
    grid = H // br
    return pl.pallas_call(
        functools.partial(_ng_pool_body, n_real),
        grid=(grid,),
        in_specs=[pl.BlockSpec((br, npad), lambda i: (i, 0))],
        out_specs=[
            pl.BlockSpec((br, npad), lambda i: (i, 0)),
            pl.BlockSpec((br, 128), lambda i: (i, 0)),
        ],
        out_shape=[
            jax.ShapeDtypeStruct((H, npad), F32),
            jax.ShapeDtypeStruct((H, 128), F32),
        ],
    )(yT)


def _head_body(xl_ref, w1_ref, w2_ref, pw_ref, pb_ref, out_ref):
    xl = xl_ref[...]
    h = xl @ w1_ref[...]
    m = jnp.mean(h, axis=0, keepdims=True)
    v = jnp.mean((h - m) ** 2, axis=0, keepdims=True)
    h = _gelu((h - m) / jnp.sqrt(v + 1e-05))
    h2 = h @ w2_ref[...]
    m2 = jnp.mean(h2, axis=0, keepdims=True)
    v2 = jnp.mean((h2 - m2) ** 2, axis=0, keepdims=True)
    h2 = _gelu((h2 - m2) / jnp.sqrt(v2 + 1e-05))
    out_ref[...] = h2 @ pw_ref[...] + pb_ref[0, 0]


def _head(xl, w1, w2, pw, pb):
    nc = xl.shape[0]
    return pl.pallas_call(
        _head_body,
        out_shape=jax.ShapeDtypeStruct((nc, 1), F32),
    )(xl, w1, w2, pw.reshape(-1, 1), pb.reshape(1, 1))


def _sc_mesh():
    return plsc.VectorSubcoreMesh(core_axis_name="c", subcore_axis_name="s")


def _vrot(scratch16, x, sh):
    scratch16[...] = x
    idx = (jnp.arange(16, dtype=I32) + sh) & 15
    return plsc.load_gather(scratch16, [idx])


def _vallsum(scratch16, x):
    for sh in (8, 4, 2, 1):
        x = x + _vrot(scratch16, x, sh)
    return x


def _vallmax(scratch16, x):
    for sh in (8, 4, 2, 1):
        x = jnp.maximum(x, _vrot(scratch16, x, sh))
    return x


def _make_k1(npad, epad, e_real):
    epw = epad // NW
    c1 = 96
    cpw = epw // c1
    npairs = cpw // 2

    @functools.partial(
        pl.kernel,
        mesh=_sc_mesh(),
        compiler_params=pltpu.CompilerParams(needs_layout_passes=False),
        out_type=(
            jax.ShapeDtypeStruct((epad,), F32),
            jax.ShapeDtypeStruct((NW, 16), F32),
        ),
        scratch_types=[
            pltpu.VMEM((c1,), I32), pltpu.VMEM((c1,), I32),
            pltpu.VMEM((c1,), I32), pltpu.VMEM((c1,), I32),
            pltpu.VMEM((c1, H), F32), pltpu.VMEM((c1, H), F32),
            pltpu.VMEM((c1, H), F32), pltpu.VMEM((c1, H), F32),
            pltpu.VMEM((c1,), F32),
            pltpu.VMEM((H,), F32),
            pltpu.VMEM((16,), F32),
            pltpu.SemaphoreType.DMA, pltpu.SemaphoreType.DMA,
            pltpu.SemaphoreType.DMA, pltpu.SemaphoreType.DMA,
        ],
    )
    def k1(xl_hbm, xr_hbm, srcs_hbm, dsts_hbm, att_hbm, lg_hbm, wmax_hbm,
           is0, id0, is1, id1, rl0, rr0, rl1, rr1, lg_v, att_v, wv,
           sl0, sr0, sl1, sr1):
        wid = lax.axis_index("s") * 2 + lax.axis_index("c")
        base_w = wid * epw
        pltpu.sync_copy(att_hbm, att_v)
        att_regs = [att_v[pl.ds(16 * j, 16)] for j in range(16)]
        iota16 = jnp.arange(16, dtype=I32)
        negv = jnp.full((16,), NEG, F32)

        def issue(cb, idx_s, idx_d, rl, rr, sl, sr):
            pltpu.sync_copy(srcs_hbm.at[pl.ds(cb, c1)], idx_s)
            pltpu.sync_copy(dsts_hbm.at[pl.ds(cb, c1)], idx_d)
            pltpu.async_copy(xl_hbm.at[idx_s], rl, sl)
            pltpu.async_copy(xr_hbm.at[idx_d], rr, sr)

        def waitb(idx_s, idx_d, rl, rr, sl, sr):
            pltpu.make_async_copy(xl_hbm.at[idx_s], rl, sl).wait()
            pltpu.make_async_copy(xr_hbm.at[idx_d], rr, sr).wait()

        def compute(cb, rl, rr, mx):
            for k in range(c1 // 16):
                lg_v[pl.ds(16 * k, 16)] = jnp.zeros((16,), F32)

            def edge_body(i, _):
                accs = [jnp.zeros((16,), F32) for _ in range(4)]
                for j in range(16):
                    t = rl[i, pl.ds(16 * j, 16)] + rr[i, pl.ds(16 * j, 16)]
                    t = jnp.maximum(t, 0.2 * t)
                    accs[j % 4] = accs[j % 4] + att_regs[j] * t
                acc = (accs[0] + accs[1]) + (accs[2] + accs[3])
                plsc.addupdate_scatter(lg_v, [jnp.full((16,), i, I32)], acc)
                return 0

            lax.fori_loop(0, c1, edge_body, 0, unroll=2)
            for k in range(c1 // 16):
                l16 = lg_v[pl.ds(16 * k, 16)]
                eid = jnp.full((16,), cb + 16 * k, I32) + iota16
                l16 = jnp.where(eid < jnp.full((16,), e_real, I32),
                                l16, negv)
                lg_v[pl.ds(16 * k, 16)] = l16
                mx = jnp.maximum(mx, l16)
            pltpu.sync_copy(lg_v, lg_hbm.at[pl.ds(cb, c1)])
            return mx

        issue(base_w, is0, id0, rl0, rr0, sl0, sr0)

        def pair_body(p, mx):
            cb0 = base_w + (2 * p) * c1
            cb1 = cb0 + c1
            issue(cb1, is1, id1, rl1, rr1, sl1, sr1)
            waitb(is0, id0, rl0, rr0, sl0, sr0)
            mx = compute(cb0, rl0, rr0, mx)

            @pl.when(p + 1 < npairs)
            def _():
                issue(cb0 + 2 * c1, is0, id0, rl0, rr0, sl0, sr0)

            waitb(is1, id1, rl1, rr1, sl1, sr1)
            mx = compute(cb1, rl1, rr1, mx)
            return mx

        mx = lax.fori_loop(0, npairs, pair_body, negv)
        wv[...] = mx
        pltpu.sync_copy(wv, wmax_hbm.at[wid])

    return k1


def _make_k2(npad, epad):
    epw = epad // NW
    cpw = epw // CHUNK
    nseg = npad // 16

    @functools.partial(
        pl.kernel,
        mesh=_sc_mesh(),
        compiler_params=pltpu.CompilerParams(needs_layout_passes=False),
        out_type=jax.ShapeDtypeStruct((2, npad), F32),
        scratch_types=[
            pltpu.VMEM((npad,), F32),
            pltpu.VMEM((CHUNK,), I32),
            pltpu.VMEM((CHUNK,), F32),
            pltpu.VMEM((NW, 16), F32),
            pltpu.VMEM((nseg,), F32),
            pltpu.VMEM((nseg,), F32),
            pltpu.VMEM((16,), F32),
            pltpu.VMEM_SHARED((16, npad), F32),
        ],
    )
    def k2(lg_hbm, dsts_hbm, wmax_hbm, den_hbm,
           den_v, idx_d, lg_v, wmax_v, tmp_v, acc_v, bounce, den_sh):
        cid = lax.axis_index("c")
        sid = lax.axis_index("s")
        wid = sid * 2 + cid
        pltpu.sync_copy(wmax_hbm, wmax_v)
        g = jnp.full((16,), NEG, F32)
        for w in range(NW):
            g = jnp.maximum(g, wmax_v[w, :])
        gmax = _vallmax(bounce, g)

        def zero_body(i, _):
            den_v[pl.ds(i * 16, 16)] = jnp.zeros((16,), F32)
            return 0

        lax.fori_loop(0, npad // 16, zero_body, 0)

        def chunk_body(c, _):
            cb = wid * epw + c * CHUNK
            pltpu.sync_copy(dsts_hbm.at[pl.ds(cb, CHUNK)], idx_d)
            pltpu.sync_copy(lg_hbm.at[pl.ds(cb, CHUNK)], lg_v)
            for k in range(CHUNK // 16):
                d16 = idx_d[pl.ds(16 * k, 16)]
                ex = jnp.exp(lg_v[pl.ds(16 * k, 16)] - gmax)
                plsc.addupdate_scatter(den_v, [d16], ex)
            return 0

        lax.fori_loop(0, cpw, chunk_body, 0)

        pltpu.sync_copy(den_v, den_sh.at[sid])
        plsc.subcore_barrier()
        nb = sid * nseg

        def acc_zero(i, _):
            acc_v[pl.ds(i * 16, 16)] = jnp.zeros((16,), F32)
            return 0

        lax.fori_loop(0, nseg // 16, acc_zero, 0)
        for w in range(16):
            pltpu.sync_copy(den_sh.at[w, pl.ds(nb, nseg)], tmp_v)

            def add_body(i, _):
                acc_v[pl.ds(i * 16, 16)] = (acc_v[pl.ds(i * 16, 16)]
                                            + tmp_v[pl.ds(i * 16, 16)])
                return 0

            lax.fori_loop(0, nseg // 16, add_body, 0)
        pltpu.sync_copy(acc_v, den_hbm.at[cid, pl.ds(nb, nseg)])

    return k2


def _make_k3(npad, epad):
    nchunks = epad // CHUNK
    CPT = 4

    @functools.partial(
        pl.kernel,
        mesh=_sc_mesh(),
        compiler_params=pltpu.CompilerParams(needs_layout_passes=False),
        out_type=jax.ShapeDtypeStruct((H, npad), F32),
        scratch_types=[
            pltpu.VMEM((CPT, npad), F32),
            pltpu.VMEM((CPT, npad), F32),
            pltpu.VMEM((npad,), F32),
            pltpu.VMEM((npad,), F32),
            pltpu.VMEM((CHUNK,), I32), pltpu.VMEM((CHUNK,), I32),
            pltpu.VMEM((CHUNK,), I32), pltpu.VMEM((CHUNK,), I32),
            pltpu.VMEM((CHUNK,), F32), pltpu.VMEM((CHUNK,), F32),
            pltpu.VMEM((NW, 16), F32),
            pltpu.VMEM((16,), F32),
            pltpu.SemaphoreType.DMA, pltpu.SemaphoreType.DMA,
        ],
    )
    def k3(xlT_hbm, srcs_hbm, dsts_hbm, lg_hbm, wmax_hbm, den_hbm, outT_hbm,
           xl4, acc4, den_v, den_tmp, is0, id0, is1, id1, lgv0, lgv1,
           wmax_v, bounce, sem0, sem1):
        wid = lax.axis_index("s") * 2 + lax.axis_index("c")
        pltpu.sync_copy(wmax_hbm, wmax_v)
        g = jnp.full((16,), NEG, F32)
        for w in range(NW):
            g = jnp.maximum(g, wmax_v[w, :])
        gmax = _vallmax(bounce, g)

        pltpu.sync_copy(den_hbm.at[0], den_v)
        pltpu.sync_copy(den_hbm.at[1], den_tmp)

        def den_add(i, _):
            den_v[pl.ds(i * 16, 16)] = (den_v[pl.ds(i * 16, 16)]
                                        + den_tmp[pl.ds(i * 16, 16)])
            return 0

        lax.fori_loop(0, npad // 16, den_add, 0)

        def issue(cb, bs, bd, blg, sem):
            pltpu.async_copy(srcs_hbm.at[pl.ds(cb, CHUNK)], bs, sem)
            pltpu.async_copy(dsts_hbm.at[pl.ds(cb, CHUNK)], bd, sem)
            pltpu.async_copy(lg_hbm.at[pl.ds(cb, CHUNK)], blg, sem)

        def waitb(cb, bs, bd, blg, sem):
            pltpu.make_async_copy(srcs_hbm.at[pl.ds(cb, CHUNK)], bs,
                                  sem).wait()
            pltpu.make_async_copy(dsts_hbm.at[pl.ds(cb, CHUNK)], bd,
                                  sem).wait()
            pltpu.make_async_copy(lg_hbm.at[pl.ds(cb, CHUNK)], blg,
                                  sem).wait()

        npairs = nchunks // 2

        for rnd in range(H // (NW * CPT)):
            cols = (rnd * NW + wid) * CPT

            for j in range(CPT):
                pltpu.sync_copy(xlT_hbm.at[cols + j], xl4.at[j])

            def acc_zero(i, _):
                for j in range(CPT):
                    acc4[j, pl.ds(i * 16, 16)] = jnp.zeros((16,), F32)
                return 0

            lax.fori_loop(0, npad // 16, acc_zero, 0)

            def compute(bs, bd, blg):
                for k in range(CHUNK // 16):
                    s16 = bs[pl.ds(16 * k, 16)]
                    d16 = bd[pl.ds(16 * k, 16)]
                    ex = jnp.exp(blg[pl.ds(16 * k, 16)] - gmax)
                    dd = plsc.load_gather(den_v, [d16])
                    a16 = ex / (dd + 1e-16)
                    for j in range(CPT):
                        j16 = jnp.full((16,), j, I32)
                        v = plsc.load_gather(xl4, [j16, s16])
                        plsc.addupdate_scatter(acc4, [j16, d16], a16 * v)

            issue(0, is0, id0, lgv0, sem0)

            def pair_body(p, _):
                cb0 = (2 * p) * CHUNK
                cb1 = cb0 + CHUNK
                issue(cb1, is1, id1, lgv1, sem1)
                waitb(cb0, is0, id0, lgv0, sem0)
                compute(is0, id0, lgv0)

                @pl.when(p + 1 < npairs)
                def _():
                    issue(cb0 + 2 * CHUNK, is0, id0, lgv0, sem0)

                waitb(cb1, is1, id1, lgv1, sem1)
                compute(is1, id1, lgv1)
                return 0

            lax.fori_loop(0, npairs, pair_body, 0)

            for j in range(CPT):
                pltpu.sync_copy(acc4.at[j], outT_hbm.at[cols + j])

    return k3


def kernel(node_feat, node_opcode, edge_index, config_feat, params):
    n = node_feat.shape[0]
    e = edge_index.shape[1]
    npad = ((n + 1023) // 1024) * 1024
    bn = 1024
    e_real = e + n
    epad = ((e_real + NW * CHUNK - 1) // (NW * CHUNK)) * (NW * CHUNK)

    loops = jnp.arange(n, dtype=edge_index.dtype)
    zpad = jnp.zeros((epad - e_real,), I32)
    srcs = jnp.concatenate([edge_index[0], loops, zpad])
    dsts = jnp.concatenate([edge_index[1], loops, zpad])

    nfTn = ((node_feat - params['nf_mean']) / (params['nf_std'] + 0.0001)).T
    nfTn = jnp.pad(nfTn, ((0, 0), (0, npad - n)))
    op2d = jnp.broadcast_to(node_opcode[None, :], (8, n)).astype(I32)
    op2d = jnp.pad(op2d, ((0, 0), (0, npad - n)), constant_values=-1)
    embT = params['embed'].T

    e1T = _enc1(embT, op2d, nfTn, params['eW1'], npad, bn)
    x1T = _ng(e1T, n, npad)
    e2T = _mmT(params['eW2'], x1T, npad, bn)
    xT = _ng(e2T, n, npad)

    k1 = _make_k1(npad, epad, e_real)
    k2 = _make_k2(npad, epad)
    k3 = _make_k3(npad, epad)
    for li, gp in enumerate(params['gat']):
        xl, xr, xlT = _gat_mm(xT, gp['Wl'], gp['Wr'], gp['bl'], gp['br'],
                              npad, bn)
        lg, wmax = k1(xl, xr, srcs, dsts, gp['att'])
        den = k2(lg, dsts, wmax)
        outT = k3(xlT, srcs, dsts, lg, wmax, den)
        if li < 3:
            xT = _ng(outT, n, npad)
        else:
            xT, pool2d = _ng_pool(outT, n, npad)

    pool = pool2d[:, 0]

    cf = (config_feat - params['cf_mean']) / (params['cf_std'] + 0.0001)
    xl_head = jnp.concatenate(
        [cf, jnp.broadcast_to(pool[None, :], (cf.shape[0], H))], axis=1)
    runtime = _head(xl_head, params['lW1'], params['lW2'],
                    params['pW'], params['pb'])
    return runtime[:, 0]


# speedup vs baseline: 3.3206x; 1.0204x over previous
import functools

import jax
import jax.numpy as jnp
from jax import lax
from jax.experimental import pallas as pl
from jax.experimental.pallas import tpu as pltpu
from jax.experimental.pallas import tpu_sc as plsc

F32 = jnp.float32
I32 = jnp.int32

H = 256
NW = 32
CHUNK = 128
NEG = -1e30


def _erf(x):
    s = jnp.sign(x)
    a = jnp.abs(x)
    t = 1.0 / (1.0 + 0.3275911 * a)
    poly = t * (0.254829592 + t * (-0.284496736 + t * (1.421413741
           + t * (-1.453152027 + t * 1.061405429))))
    return s * (1.0 - poly * jnp.exp(-a * a))


def _gelu(x):
    return 0.5 * x * (1.0 + _erf(x * 0.7071067811865476))


def _enc1_body(embT_ref, op_ref, nfT_ref, w1_ref, e1T_ref):
    embT = embT_ref[...]
    nrm = jnp.sqrt(jnp.sum(embT * embT, axis=0, keepdims=True))
    scale = jnp.minimum(1.0, 1.0 / jnp.maximum(nrm, 1e-07))
    embT = embT * scale
    op_row = op_ref[0:1, :]
    oh = (lax.broadcasted_iota(I32, (embT.shape[1], op_row.shape[1]), 0)
          == op_row).astype(F32)
    x0_emb = lax.dot_general(embT, oh, (((1,), (0,)), ((), ())),
                             preferred_element_type=F32)
    x0 = jnp.concatenate([x0_emb, nfT_ref[...]], axis=0)
    e1T_ref[...] = lax.dot_general(w1_ref[...], x0, (((0,), (0,)), ((), ())),
                                   preferred_element_type=F32)


def _enc1(embT, op2d, nfTn, w1, npad, bn):
    grid = npad // bn
    return pl.pallas_call(
        _enc1_body,
        grid=(grid,),
        in_specs=[
            pl.BlockSpec(embT.shape, lambda i: (0, 0)),
            pl.BlockSpec((8, bn), lambda i: (0, i)),
            pl.BlockSpec((140, bn), lambda i: (0, i)),
            pl.BlockSpec(w1.shape, lambda i: (0, 0)),
        ],
        out_specs=pl.BlockSpec((H, bn), lambda i: (0, i)),
        out_shape=jax.ShapeDtypeStruct((H, npad), F32),
    )(embT, op2d, nfTn, w1)


def _mmT_body(w_ref, xT_ref, oT_ref):
    oT_ref[...] = lax.dot_general(w_ref[...], xT_ref[...],
                                  (((0,), (0,)), ((), ())),
                                  preferred_element_type=F32)


def _mmT(w, xT, npad, bn):
    grid = npad // bn
    return pl.pallas_call(
        _mmT_body,
        grid=(grid,),
        in_specs=[
            pl.BlockSpec(w.shape, lambda i: (0, 0)),
            pl.BlockSpec((w.shape[0], bn), lambda i: (0, i)),
        ],
        out_specs=pl.BlockSpec((w.shape[1], bn), lambda i: (0, i)),
        out_shape=jax.ShapeDtypeStruct((w.shape[1], npad), F32),
    )(w, xT)


def _gat_mm_body(xT_ref, wl_ref, wr_ref, bl_ref, br_ref, blc_ref,
                 xl_ref, xr_ref, xlT_ref):
    xTb = xT_ref[...]
    wl = wl_ref[...]
    wr = wr_ref[...]
    xl_ref[...] = lax.dot_general(xTb, wl, (((0,), (0,)), ((), ())),
                                  preferred_element_type=F32) + bl_ref[...]
    xr_ref[...] = lax.dot_general(xTb, wr, (((0,), (0,)), ((), ())),
                                  preferred_element_type=F32) + br_ref[...]
    xlT_ref[...] = lax.dot_general(wl, xTb, (((0,), (0,)), ((), ())),
                                   preferred_element_type=F32) + blc_ref[...]


def _gat_mm(xT, wl, wr, bl, br, npad, bn):
    grid = npad // bn
    return pl.pallas_call(
        _gat_mm_body,
        grid=(grid,),
        in_specs=[
            pl.BlockSpec((H, bn), lambda i: (0, i)),
            pl.BlockSpec((H, H), lambda i: (0, 0)),
            pl.BlockSpec((H, H), lambda i: (0, 0)),
            pl.BlockSpec((1, H), lambda i: (0, 0)),
            pl.BlockSpec((1, H), lambda i: (0, 0)),
            pl.BlockSpec((H, 1), lambda i: (0, 0)),
        ],
        out_specs=[
            pl.BlockSpec((bn, H), lambda i: (i, 0)),
            pl.BlockSpec((bn, H), lambda i: (i, 0)),
            pl.BlockSpec((H, bn), lambda i: (0, i)),
        ],
        out_shape=[
            jax.ShapeDtypeStruct((npad, H), F32),
            jax.ShapeDtypeStruct((npad, H), F32),
            jax.ShapeDtypeStruct((H, npad), F32),
        ],
    )(xT, wl, wr, bl.reshape(1, H), br.reshape(1, H), bl.reshape(H, 1))


def _ng_body(n_real, yT_ref, oT_ref):
    y = yT_ref[...]
    npad = y.shape[1]
    mask = (lax.broadcasted_iota(I32, (1, npad), 1) < n_real).astype(F32)
    s = jnp.sum(y * mask, axis=1, keepdims=True)
    m = s * (1.0 / n_real)
    d = (y - m) * mask
    v = jnp.sum(d * d, axis=1, keepdims=True) * (1.0 / n_real)
    oT_ref[...] = _gelu((y - m) / jnp.sqrt(v + 1e-05))


def _ng(yT, n_real, npad, br=64):
    grid = H // br
    return pl.pallas_call(
        functools.partial(_ng_body, n_real),
        grid=(grid,),
        in_specs=[pl.BlockSpec((br, npad), lambda i: (i, 0))],
        out_specs=pl.BlockSpec((br, npad), lambda i: (i, 0)),
        out_shape=jax.ShapeDtypeStruct((H, npad), F32),
    )(yT)


def _ng_pool_body(n_real, yT_ref, oT_ref, pool_ref):
    y = yT_ref[...]
    npad = y.shape[1]
    maskb = lax.broadcasted_iota(I32, (1, npad), 1) < n_real
    mask = maskb.astype(F32)
    s = jnp.sum(y * mask, axis=1, keepdims=True)
    m = s * (1.0 / n_real)
    d = (y - m) * mask
    v = jnp.sum(d * d, axis=1, keepdims=True) * (1.0 / n_real)
    out = _gelu((y - m) / jnp.sqrt(v + 1e-05))
    oT_ref[...] = out
    pmean = jnp.sum(out * mask, axis=1, keepdims=True) * (1.0 / n_real)
    pmax = jnp.max(jnp.where(maskb, out, NEG), axis=1, keepdims=True)
    pool_ref[...] = jnp.broadcast_to(pmean + pmax, (out.shape[0], 128))


def _ng_pool(yT, n_real, npad, br=64):
    grid = H // br
    return pl.pallas_call(
        functools.partial(_ng_pool_body, n_real),
        grid=(grid,),
        in_specs=[pl.BlockSpec((br, npad), lambda i: (i, 0))],
        out_specs=[
            pl.BlockSpec((br, npad), lambda i: (i, 0)),
            pl.BlockSpec((br, 128), lambda i: (i, 0)),
        ],
        out_shape=[
            jax.ShapeDtypeStruct((H, npad), F32),
            jax.ShapeDtypeStruct((H, 128), F32),
        ],
    )(yT)


def _head_body(xl_ref, w1_ref, w2_ref, pw_ref, pb_ref, out_ref):
    xl = xl_ref[...]
    h = xl @ w1_ref[...]
    m = jnp.mean(h, axis=0, keepdims=True)
    v = jnp.mean((h - m) ** 2, axis=0, keepdims=True)
    h = _gelu((h - m) / jnp.sqrt(v + 1e-05))
    h2 = h @ w2_ref[...]
    m2 = jnp.mean(h2, axis=0, keepdims=True)
    v2 = jnp.mean((h2 - m2) ** 2, axis=0, keepdims=True)
    h2 = _gelu((h2 - m2) / jnp.sqrt(v2 + 1e-05))
    out_ref[...] = h2 @ pw_ref[...] + pb_ref[0, 0]


def _head(xl, w1, w2, pw, pb):
    nc = xl.shape[0]
    return pl.pallas_call(
        _head_body,
        out_shape=jax.ShapeDtypeStruct((nc, 1), F32),
    )(xl, w1, w2, pw.reshape(-1, 1), pb.reshape(1, 1))


def _sc_mesh():
    return plsc.VectorSubcoreMesh(core_axis_name="c", subcore_axis_name="s")


def _vrot(scratch16, x, sh):
    scratch16[...] = x
    idx = (jnp.arange(16, dtype=I32) + sh) & 15
    return plsc.load_gather(scratch16, [idx])


def _vallsum(scratch16, x):
    for sh in (8, 4, 2, 1):
        x = x + _vrot(scratch16, x, sh)
    return x


def _vallmax(scratch16, x):
    for sh in (8, 4, 2, 1):
        x = jnp.maximum(x, _vrot(scratch16, x, sh))
    return x


def _make_k1(npad, epad, e_real):
    epw = epad // NW
    c1 = 96
    cpw = epw // c1
    npairs = cpw // 2

    @functools.partial(
        pl.kernel,
        mesh=_sc_mesh(),
        compiler_params=pltpu.CompilerParams(needs_layout_passes=False),
        out_type=(
            jax.ShapeDtypeStruct((epad,), F32),
            jax.ShapeDtypeStruct((NW, 16), F32),
        ),
        scratch_types=[
            pltpu.VMEM((c1,), I32), pltpu.VMEM((c1,), I32),
            pltpu.VMEM((c1,), I32), pltpu.VMEM((c1,), I32),
            pltpu.VMEM((c1, H), F32), pltpu.VMEM((c1, H), F32),
            pltpu.VMEM((c1, H), F32), pltpu.VMEM((c1, H), F32),
            pltpu.VMEM((c1,), F32),
            pltpu.VMEM((H,), F32),
            pltpu.VMEM((16,), F32),
            pltpu.SemaphoreType.DMA, pltpu.SemaphoreType.DMA,
            pltpu.SemaphoreType.DMA, pltpu.SemaphoreType.DMA,
        ],
    )
    def k1(xl_hbm, xr_hbm, srcs_hbm, dsts_hbm, att_hbm, lg_hbm, wmax_hbm,
           is0, id0, is1, id1, rl0, rr0, rl1, rr1, lg_v, att_v, wv,
           sl0, sr0, sl1, sr1):
        wid = lax.axis_index("s") * 2 + lax.axis_index("c")
        base_w = wid * epw
        pltpu.sync_copy(att_hbm, att_v)
        att_regs = [att_v[pl.ds(16 * j, 16)] for j in range(16)]
        iota16 = jnp.arange(16, dtype=I32)
        negv = jnp.full((16,), NEG, F32)

        def issue(cb, idx_s, idx_d, rl, rr, sl, sr):
            pltpu.sync_copy(srcs_hbm.at[pl.ds(cb, c1)], idx_s)
            pltpu.sync_copy(dsts_hbm.at[pl.ds(cb, c1)], idx_d)
            pltpu.async_copy(xl_hbm.at[idx_s], rl, sl)
            pltpu.async_copy(xr_hbm.at[idx_d], rr, sr)

        def waitb(idx_s, idx_d, rl, rr, sl, sr):
            pltpu.make_async_copy(xl_hbm.at[idx_s], rl, sl).wait()
            pltpu.make_async_copy(xr_hbm.at[idx_d], rr, sr).wait()

        def compute(cb, rl, rr, mx):
            for k in range(c1 // 16):
                lg_v[pl.ds(16 * k, 16)] = jnp.zeros((16,), F32)

            @plsc.parallel_loop(0, c1, unroll=4)
            def _edge_body(i):
                accs = [jnp.zeros((16,), F32) for _ in range(4)]
                for j in range(16):
                    t = rl[i, pl.ds(16 * j, 16)] + rr[i, pl.ds(16 * j, 16)]
                    t = jnp.maximum(t, 0.2 * t)
                    accs[j % 4] = accs[j % 4] + att_regs[j] * t
                acc = (accs[0] + accs[1]) + (accs[2] + accs[3])
                plsc.addupdate_scatter(lg_v, [jnp.full((16,), i, I32)], acc)
            for k in range(c1 // 16):
                l16 = lg_v[pl.ds(16 * k, 16)]
                eid = jnp.full((16,), cb + 16 * k, I32) + iota16
                l16 = jnp.where(eid < jnp.full((16,), e_real, I32),
                                l16, negv)
                lg_v[pl.ds(16 * k, 16)] = l16
                mx = jnp.maximum(mx, l16)
            pltpu.sync_copy(lg_v, lg_hbm.at[pl.ds(cb, c1)])
            return mx

        issue(base_w, is0, id0, rl0, rr0, sl0, sr0)

        def pair_body(p, mx):
            cb0 = base_w + (2 * p) * c1
            cb1 = cb0 + c1
            issue(cb1, is1, id1, rl1, rr1, sl1, sr1)
            waitb(is0, id0, rl0, rr0, sl0, sr0)
            mx = compute(cb0, rl0, rr0, mx)

            @pl.when(p + 1 < npairs)
            def _():
                issue(cb0 + 2 * c1, is0, id0, rl0, rr0, sl0, sr0)

            waitb(is1, id1, rl1, rr1, sl1, sr1)
            mx = compute(cb1, rl1, rr1, mx)
            return mx

        mx = lax.fori_loop(0, npairs, pair_body, negv)
        wv[...] = mx
        pltpu.sync_copy(wv, wmax_hbm.at[wid])

    return k1


def _make_k2(npad, epad):
    epw = epad // NW
    cpw = epw // CHUNK
    nseg = npad // 16

    @functools.partial(
        pl.kernel,
        mesh=_sc_mesh(),
        compiler_params=pltpu.CompilerParams(needs_layout_passes=False),
        out_type=jax.ShapeDtypeStruct((2, npad), F32),
        scratch_types=[
            pltpu.VMEM((npad,), F32),
            pltpu.VMEM((CHUNK,), I32),
            pltpu.VMEM((CHUNK,), F32),
            pltpu.VMEM((NW, 16), F32),
            pltpu.VMEM((nseg,), F32),
            pltpu.VMEM((nseg,), F32),
            pltpu.VMEM((16,), F32),
            pltpu.VMEM_SHARED((16, npad), F32),
        ],
    )
    def k2(lg_hbm, dsts_hbm, wmax_hbm, den_hbm,
           den_v, idx_d, lg_v, wmax_v, tmp_v, acc_v, bounce, den_sh):
        cid = lax.axis_index("c")
        sid = lax.axis_index("s")
        wid = sid * 2 + cid
        pltpu.sync_copy(wmax_hbm, wmax_v)
        g = jnp.full((16,), NEG, F32)
        for w in range(NW):
            g = jnp.maximum(g, wmax_v[w, :])
        gmax = _vallmax(bounce, g)

        def zero_body(i, _):
            den_v[pl.ds(i * 16, 16)] = jnp.zeros((16,), F32)
            return 0

        lax.fori_loop(0, npad // 16, zero_body, 0)

        def chunk_body(c, _):
            cb = wid * epw + c * CHUNK
            pltpu.sync_copy(dsts_hbm.at[pl.ds(cb, CHUNK)], idx_d)
            pltpu.sync_copy(lg_hbm.at[pl.ds(cb, CHUNK)], lg_v)
            for k in range(CHUNK // 16):
                d16 = idx_d[pl.ds(16 * k, 16)]
                ex = jnp.exp(lg_v[pl.ds(16 * k, 16)] - gmax)
                plsc.addupdate_scatter(den_v, [d16], ex)
            return 0

        lax.fori_loop(0, cpw, chunk_body, 0)

        pltpu.sync_copy(den_v, den_sh.at[sid])
        plsc.subcore_barrier()
        nb = sid * nseg

        def acc_zero(i, _):
            acc_v[pl.ds(i * 16, 16)] = jnp.zeros((16,), F32)
            return 0

        lax.fori_loop(0, nseg // 16, acc_zero, 0)
        for w in range(16):
            pltpu.sync_copy(den_sh.at[w, pl.ds(nb, nseg)], tmp_v)

            def add_body(i, _):
                acc_v[pl.ds(i * 16, 16)] = (acc_v[pl.ds(i * 16, 16)]
                                            + tmp_v[pl.ds(i * 16, 16)])
                return 0

            lax.fori_loop(0, nseg // 16, add_body, 0)
        pltpu.sync_copy(acc_v, den_hbm.at[cid, pl.ds(nb, nseg)])

    return k2


def _make_k3(npad, epad):
    nchunks = epad // CHUNK
    CPT = 4

    @functools.partial(
        pl.kernel,
        mesh=_sc_mesh(),
        compiler_params=pltpu.CompilerParams(needs_layout_passes=False),
        out_type=jax.ShapeDtypeStruct((H, npad), F32),
        scratch_types=[
            pltpu.VMEM((CPT, npad), F32),
            pltpu.VMEM((CPT, npad), F32),
            pltpu.VMEM((npad,), F32),
            pltpu.VMEM((npad,), F32),
            pltpu.VMEM((CHUNK,), I32), pltpu.VMEM((CHUNK,), I32),
            pltpu.VMEM((CHUNK,), I32), pltpu.VMEM((CHUNK,), I32),
            pltpu.VMEM((CHUNK,), F32), pltpu.VMEM((CHUNK,), F32),
            pltpu.VMEM((NW, 16), F32),
            pltpu.VMEM((16,), F32),
            pltpu.SemaphoreType.DMA, pltpu.SemaphoreType.DMA,
        ],
    )
    def k3(xlT_hbm, srcs_hbm, dsts_hbm, lg_hbm, wmax_hbm, den_hbm, outT_hbm,
           xl4, acc4, den_v, den_tmp, is0, id0, is1, id1, lgv0, lgv1,
           wmax_v, bounce, sem0, sem1):
        wid = lax.axis_index("s") * 2 + lax.axis_index("c")
        pltpu.sync_copy(wmax_hbm, wmax_v)
        g = jnp.full((16,), NEG, F32)
        for w in range(NW):
            g = jnp.maximum(g, wmax_v[w, :])
        gmax = _vallmax(bounce, g)

        pltpu.sync_copy(den_hbm.at[0], den_v)
        pltpu.sync_copy(den_hbm.at[1], den_tmp)

        def den_add(i, _):
            den_v[pl.ds(i * 16, 16)] = (den_v[pl.ds(i * 16, 16)]
                                        + den_tmp[pl.ds(i * 16, 16)])
            return 0

        lax.fori_loop(0, npad // 16, den_add, 0)

        def issue(cb, bs, bd, blg, sem):
            pltpu.async_copy(srcs_hbm.at[pl.ds(cb, CHUNK)], bs, sem)
            pltpu.async_copy(dsts_hbm.at[pl.ds(cb, CHUNK)], bd, sem)
            pltpu.async_copy(lg_hbm.at[pl.ds(cb, CHUNK)], blg, sem)

        def waitb(cb, bs, bd, blg, sem):
            pltpu.make_async_copy(srcs_hbm.at[pl.ds(cb, CHUNK)], bs,
                                  sem).wait()
            pltpu.make_async_copy(dsts_hbm.at[pl.ds(cb, CHUNK)], bd,
                                  sem).wait()
            pltpu.make_async_copy(lg_hbm.at[pl.ds(cb, CHUNK)], blg,
                                  sem).wait()

        npairs = nchunks // 2

        for rnd in range(H // (NW * CPT)):
            cols = (rnd * NW + wid) * CPT

            for j in range(CPT):
                pltpu.sync_copy(xlT_hbm.at[cols + j], xl4.at[j])

            def acc_zero(i, _):
                for j in range(CPT):
                    acc4[j, pl.ds(i * 16, 16)] = jnp.zeros((16,), F32)
                return 0

            lax.fori_loop(0, npad // 16, acc_zero, 0)

            def compute(bs, bd, blg):
                for k in range(CHUNK // 16):
                    s16 = bs[pl.ds(16 * k, 16)]
                    d16 = bd[pl.ds(16 * k, 16)]
                    ex = jnp.exp(blg[pl.ds(16 * k, 16)] - gmax)
                    dd = plsc.load_gather(den_v, [d16])
                    a16 = ex / (dd + 1e-16)
                    for j in range(CPT):
                        j16 = jnp.full((16,), j, I32)
                        v = plsc.load_gather(xl4, [j16, s16])
                        plsc.addupdate_scatter(acc4, [j16, d16], a16 * v)

            issue(0, is0, id0, lgv0, sem0)

            def pair_body(p, _):
                cb0 = (2 * p) * CHUNK
                cb1 = cb0 + CHUNK
                issue(cb1, is1, id1, lgv1, sem1)
                waitb(cb0, is0, id0, lgv0, sem0)
                compute(is0, id0, lgv0)

                @pl.when(p + 1 < npairs)
                def _():
                    issue(cb0 + 2 * CHUNK, is0, id0, lgv0, sem0)

                waitb(cb1, is1, id1, lgv1, sem1)
                compute(is1, id1, lgv1)
                return 0

            lax.fori_loop(0, npairs, pair_body, 0)

            for j in range(CPT):
                pltpu.sync_copy(acc4.at[j], outT_hbm.at[cols + j])

    return k3


def kernel(node_feat, node_opcode, edge_index, config_feat, params):
    n = node_feat.shape[0]
    e = edge_index.shape[1]
    npad = ((n + 1023) // 1024) * 1024
    bn = 1024
    e_real = e + n
    epad = ((e_real + NW * CHUNK - 1) // (NW * CHUNK)) * (NW * CHUNK)

    loops = jnp.arange(n, dtype=edge_index.dtype)
    zpad = jnp.zeros((epad - e_real,), I32)
    srcs = jnp.concatenate([edge_index[0], loops, zpad])
    dsts = jnp.concatenate([edge_index[1], loops, zpad])

    nfTn = ((node_feat - params['nf_mean']) / (params['nf_std'] + 0.0001)).T
    nfTn = jnp.pad(nfTn, ((0, 0), (0, npad - n)))
    op2d = jnp.broadcast_to(node_opcode[None, :], (8, n)).astype(I32)
    op2d = jnp.pad(op2d, ((0, 0), (0, npad - n)), constant_values=-1)
    embT = params['embed'].T

    e1T = _enc1(embT, op2d, nfTn, params['eW1'], npad, bn)
    x1T = _ng(e1T, n, npad)
    e2T = _mmT(params['eW2'], x1T, npad, bn)
    xT = _ng(e2T, n, npad)

    k1 = _make_k1(npad, epad, e_real)
    k2 = _make_k2(npad, epad)
    k3 = _make_k3(npad, epad)
    for li, gp in enumerate(params['gat']):
        xl, xr, xlT = _gat_mm(xT, gp['Wl'], gp['Wr'], gp['bl'], gp['br'],
                              npad, bn)
        lg, wmax = k1(xl, xr, srcs, dsts, gp['att'])
        den = k2(lg, dsts, wmax)
        outT = k3(xlT, srcs, dsts, lg, wmax, den)
        if li < 3:
            xT = _ng(outT, n, npad)
        else:
            xT, pool2d = _ng_pool(outT, n, npad)

    pool = pool2d[:, 0]

    cf = (config_feat - params['cf_mean']) / (params['cf_std'] + 0.0001)
    xl_head = jnp.concatenate(
        [cf, jnp.broadcast_to(pool[None, :], (cf.shape[0], H))], axis=1)
    runtime = _head(xl_head, params['lW1'], params['lW2'],
                    params['pW'], params['pb'])
    return runtime[:, 0]
